# baseline (speedup 1.0000x reference)
import functools

import jax
import jax.numpy as jnp
from jax import lax
from jax.experimental import pallas as pl
from jax.experimental.pallas import tpu as pltpu
from jax.experimental.pallas import tpu_sc as plsc

NC, NS, L = 2, 16, 16
NW = NC * NS
B = 1048576
N = 1024
P = B // NW
T = P // N
R = N // L

_OFF2 = (0, 256, 65536, 65792)


def _tri_body(x_hbm, grid_hbm, sig_hbm, alp_hbm,
              xv, fxv, fyv, fzv, vmv, izmv, idxv, rowsv, sigv, alpv, sem):
    wid = lax.axis_index("s") * NC + lax.axis_index("c")
    base0 = wid * P
    lane = lax.iota(jnp.int32, L)

    def chunk(t, carry):
        base = base0 + t * N
        pltpu.sync_copy(x_hbm.at[pl.ds(3 * base, 3 * N)], xv)

        def phase1(j, carry):
            pv = j * L + lane
            comps = []
            for c in range(3):
                xc = plsc.load_gather(xv, [pv * 3 + c])
                tc = xc * jnp.float32(255.0)
                tcl = jnp.minimum(jnp.maximum(tc, jnp.float32(0.0)),
                                  jnp.float32(254.0))
                ic = tcl.astype(jnp.int32)
                fc = tc - ic.astype(jnp.float32)
                comps.append((tc, ic, fc))
            (t0, i0, f0), (t1, i1, f1), (t2, i2, f2) = comps
            valid = ((t0 >= 0.0) & (t0 <= 255.0)
                     & (t1 >= 0.0) & (t1 <= 255.0)
                     & (t2 >= 0.0) & (t2 <= 255.0))
            fxv[pl.ds(j * L, L)] = f0
            fyv[pl.ds(j * L, L)] = f1
            fzv[pl.ds(j * L, L)] = f2
            vmv[pl.ds(j * L, L)] = jnp.where(valid, jnp.float32(1.0),
                                             jnp.float32(0.0))
            izmv[pl.ds(j * L, L)] = lax.bitwise_and(i2, 3)
            vbase = lax.shift_left(i0, 16) + lax.shift_left(i1, 8) + i2
            col0 = lane * 8
            for q, off in enumerate(_OFF2):
                v = vbase + off
                r0 = lax.shift_right_logical(v, 2)
                r1 = lax.shift_right_logical(v + 1, 2)
                plsc.store_scatter(idxv.at[j], [col0 + 2 * q], r0)
                plsc.store_scatter(idxv.at[j], [col0 + 2 * q + 1], r1)
            pltpu.async_copy(grid_hbm.at[idxv.at[j]],
                             rowsv.at[pl.ds(j * (8 * L), 8 * L)], sem)
            return carry

        lax.fori_loop(0, R, phase1, 0, unroll=False)
        pltpu.make_async_copy(grid_hbm.at[pl.ds(0, 8 * N)], rowsv, sem).wait()

        def phase2(j, carry):
            f0 = fxv[pl.ds(j * L, L)]
            f1 = fyv[pl.ds(j * L, L)]
            f2 = fzv[pl.ds(j * L, L)]
            vm = vmv[pl.ds(j * L, L)]
            izm = izmv[pl.ds(j * L, L)]
            qz0 = 2 * izm
            qz1 = 2 * lax.bitwise_and(izm + 1, 3)
            wx = (jnp.float32(1.0) - f0, f0)
            wy = (jnp.float32(1.0) - f1, f1)
            wz = (jnp.float32(1.0) - f2, f2)
            rbase = j * (8 * L) + lane * 8
            sig = jnp.zeros((L,), jnp.float32)
            alp = jnp.zeros((L,), jnp.float32)
            for dx in range(2):
                for dy in range(2):
                    wxy = wx[dx] * wy[dy]
                    q = dx * 2 + dy
                    for dz in range(2):
                        w = wxy * wz[dz]
                        ridx = rbase + 2 * q + dz
                        colz = qz1 if dz else qz0
                        g0 = plsc.load_gather(rowsv, [ridx, colz])
                        g1 = plsc.load_gather(rowsv, [ridx, colz + 1])
                        sig = sig + w * g0
                        alp = alp + w * g1
            sigv[pl.ds(j * L, L)] = sig * vm
            alpv[pl.ds(j * L, L)] = alp * vm
            return carry

        lax.fori_loop(0, R, phase2, 0, unroll=False)
        pltpu.sync_copy(sigv, sig_hbm.at[pl.ds(base, N)])
        pltpu.sync_copy(alpv, alp_hbm.at[pl.ds(base, N)])
        return carry

    lax.fori_loop(0, T, chunk, 0, unroll=False)


@functools.lru_cache(maxsize=None)
def _build():
    return functools.partial(
        pl.kernel,
        out_type=(jax.ShapeDtypeStruct((B,), jnp.float32),
                  jax.ShapeDtypeStruct((B,), jnp.float32)),
        mesh=plsc.VectorSubcoreMesh(core_axis_name="c", subcore_axis_name="s",
                                    num_cores=NC, num_subcores=NS),
        compiler_params=pltpu.CompilerParams(needs_layout_passes=False,
                                             use_tc_tiling_on_sc=False),
        scratch_types=[
            pltpu.VMEM((3 * N,), jnp.float32),
            pltpu.VMEM((N,), jnp.float32),
            pltpu.VMEM((N,), jnp.float32),
            pltpu.VMEM((N,), jnp.float32),
            pltpu.VMEM((N,), jnp.float32),
            pltpu.VMEM((N,), jnp.int32),
            pltpu.VMEM((R, 8 * L), jnp.int32),
            pltpu.VMEM((8 * N, 8), jnp.float32),
            pltpu.VMEM((N,), jnp.float32),
            pltpu.VMEM((N,), jnp.float32),
            pltpu.SemaphoreType.DMA,
        ],
    )(_tri_body)


def kernel(x, grid):
    xf = x.reshape(-1)
    g8 = grid.reshape(-1, 8)
    return _build()(xf, g8)


# speedup vs baseline: 35.7166x; 35.7166x over previous
import functools

import jax
import jax.numpy as jnp
from jax import lax
from jax.experimental import pallas as pl
from jax.experimental.pallas import tpu as pltpu
from jax.experimental.pallas import tpu_sc as plsc

NC, NS, L = 2, 16, 16
NW = NC * NS
B = 1048576
N = 256
P = B // NW
T = P // N
R = N // L

_DOFF = (0, 2, 512, 514)


def _phase1(t, buf, x0_hbm, x1_hbm, x2_hbm, grid_hbm, base0):
    (x0v, x1v, x2v, fxv, fyv, fzv, vmv, izmv, idxv, rowsv,
     sigv, alpv, sem) = buf
    base = base0 + t * N
    pltpu.sync_copy(x0_hbm.at[pl.ds(base, N)], x0v)
    pltpu.sync_copy(x1_hbm.at[pl.ds(base, N)], x1v)
    pltpu.sync_copy(x2_hbm.at[pl.ds(base, N)], x2v)
    lane = lax.iota(jnp.int32, L)
    rowoff = lax.shift_right_logical(lane, 3)
    colbase = lax.bitwise_and(lane, 7) * 16

    def body(j, carry):
        comps = []
        for xcv in (x0v, x1v, x2v):
            xc = xcv[pl.ds(j * L, L)]
            tc = xc * jnp.float32(255.0)
            tcl = jnp.minimum(jnp.maximum(tc, jnp.float32(0.0)),
                              jnp.float32(254.0))
            ic = tcl.astype(jnp.int32)
            fc = tc - ic.astype(jnp.float32)
            comps.append((tc, ic, fc))
        (t0, i0, f0), (t1, i1, f1), (t2, i2, f2) = comps
        valid = ((t0 >= 0.0) & (t0 <= 255.0)
                 & (t1 >= 0.0) & (t1 <= 255.0)
                 & (t2 >= 0.0) & (t2 <= 255.0))
        fxv[pl.ds(j * L, L)] = f0
        fyv[pl.ds(j * L, L)] = f1
        fzv[pl.ds(j * L, L)] = f2
        vmv[pl.ds(j * L, L)] = jnp.where(valid, jnp.float32(1.0),
                                         jnp.float32(0.0))
        izmv[pl.ds(j * L, L)] = lax.bitwise_and(i2, 7)
        bxy = lax.shift_left(i0, 9) + lax.shift_left(i1, 1)
        iz1 = i2 + 1
        zt0 = lax.shift_right_logical(i2, 7)
        zt1 = lax.shift_right_logical(iz1, 7)
        s0 = lax.shift_right_logical(lax.bitwise_and(i2, 127), 3)
        s1 = lax.shift_right_logical(lax.bitwise_and(iz1, 127), 3)
        rowvec = 2 * j + rowoff
        for q, doff in enumerate(_DOFF):
            bq = bxy + doff
            r0 = lax.shift_left(bq + zt0, 5) + s0
            r1 = lax.shift_left(bq + zt1, 5) + s1
            c16 = q * 4
            plsc.store_scatter(idxv, [rowvec, colbase + c16], r0)
            plsc.store_scatter(idxv, [rowvec, colbase + c16 + 1], r0 + 16)
            plsc.store_scatter(idxv, [rowvec, colbase + c16 + 2], r1)
            plsc.store_scatter(idxv, [rowvec, colbase + c16 + 3], r1 + 16)
        pltpu.async_copy(grid_hbm.at[idxv.at[2 * j]],
                         rowsv.at[pl.ds(j * 256, 128)], sem)
        pltpu.async_copy(grid_hbm.at[idxv.at[2 * j + 1]],
                         rowsv.at[pl.ds(j * 256 + 128, 128)], sem)
        return carry

    lax.fori_loop(0, R, body, 0, unroll=False)


def _phase2(t, buf, grid_hbm, sig_hbm, alp_hbm, base0):
    (x0v, x1v, x2v, fxv, fyv, fzv, vmv, izmv, idxv, rowsv,
     sigv, alpv, sem) = buf
    base = base0 + t * N
    pltpu.make_async_copy(grid_hbm.at[pl.ds(0, 16 * N)], rowsv, sem).wait()
    lane = lax.iota(jnp.int32, L)

    def body(j, carry):
        f0 = fxv[pl.ds(j * L, L)]
        f1 = fyv[pl.ds(j * L, L)]
        f2 = fzv[pl.ds(j * L, L)]
        vm = vmv[pl.ds(j * L, L)]
        izm = izmv[pl.ds(j * L, L)]
        col0 = izm
        col1 = lax.bitwise_and(izm + 1, 7)
        wx = (jnp.float32(1.0) - f0, f0)
        wy = (jnp.float32(1.0) - f1, f1)
        wz = (jnp.float32(1.0) - f2, f2)
        rbase = j * 256 + lane * 16
        sig = jnp.zeros((L,), jnp.float32)
        alp = jnp.zeros((L,), jnp.float32)
        for dx in range(2):
            for dy in range(2):
                wxy = wx[dx] * wy[dy]
                q4 = (dx * 2 + dy) * 4
                for dz in range(2):
                    w = wxy * wz[dz]
                    colz = col1 if dz else col0
                    ridx = rbase + q4 + 2 * dz
                    g0 = plsc.load_gather(rowsv, [ridx, colz])
                    g1 = plsc.load_gather(rowsv, [ridx + 1, colz])
                    sig = sig + w * g0
                    alp = alp + w * g1
        sigv[pl.ds(j * L, L)] = sig * vm
        alpv[pl.ds(j * L, L)] = alp * vm
        return carry

    lax.fori_loop(0, R, body, 0, unroll=False)
    pltpu.sync_copy(sigv, sig_hbm.at[pl.ds(base, N)])
    pltpu.sync_copy(alpv, alp_hbm.at[pl.ds(base, N)])


def _tri_body(x0_hbm, x1_hbm, x2_hbm, grid_hbm, sig_hbm, alp_hbm, *scr):
    bufs = (scr[0:13], scr[13:26])
    wid = lax.axis_index("s") * NC + lax.axis_index("c")
    base0 = wid * P

    _phase1(0, bufs[0], x0_hbm, x1_hbm, x2_hbm, grid_hbm, base0)

    def outer(tt, carry):
        c0 = 2 * tt
        _phase1(c0 + 1, bufs[1], x0_hbm, x1_hbm, x2_hbm, grid_hbm, base0)
        _phase2(c0, bufs[0], grid_hbm, sig_hbm, alp_hbm, base0)

        @pl.when(tt < T // 2 - 1)
        def _():
            _phase1(c0 + 2, bufs[0], x0_hbm, x1_hbm, x2_hbm, grid_hbm, base0)

        _phase2(c0 + 1, bufs[1], grid_hbm, sig_hbm, alp_hbm, base0)
        return carry

    lax.fori_loop(0, T // 2, outer, 0, unroll=False)


def _scratch_block():
    return [
        pltpu.VMEM((N,), jnp.float32),
        pltpu.VMEM((N,), jnp.float32),
        pltpu.VMEM((N,), jnp.float32),
        pltpu.VMEM((N,), jnp.float32),
        pltpu.VMEM((N,), jnp.float32),
        pltpu.VMEM((N,), jnp.float32),
        pltpu.VMEM((N,), jnp.float32),
        pltpu.VMEM((N,), jnp.int32),
        pltpu.VMEM((2 * R, 128), jnp.int32),
        pltpu.VMEM((16 * N, 8), jnp.float32),
        pltpu.VMEM((N,), jnp.float32),
        pltpu.VMEM((N,), jnp.float32),
        pltpu.SemaphoreType.DMA,
    ]


@functools.lru_cache(maxsize=None)
def _build():
    return functools.partial(
        pl.kernel,
        out_type=(jax.ShapeDtypeStruct((B,), jnp.float32),
                  jax.ShapeDtypeStruct((B,), jnp.float32)),
        mesh=plsc.VectorSubcoreMesh(core_axis_name="c", subcore_axis_name="s",
                                    num_cores=NC, num_subcores=NS),
        compiler_params=pltpu.CompilerParams(needs_layout_passes=False,
                                             use_tc_tiling_on_sc=False),
        scratch_types=_scratch_block() + _scratch_block(),
    )(_tri_body)


def kernel(x, grid):
    x0 = x[:, 0]
    x1 = x[:, 1]
    x2 = x[:, 2]
    g5 = grid.reshape(256, 256, 2, 128, 2).transpose(0, 1, 2, 4, 3)
    g8 = g5.reshape(4194304, 8)
    return _build()(x0, x1, x2, g8)
